# baseline (speedup 1.0000x reference)
import functools

import jax
import jax.numpy as jnp
from jax import lax
from jax.experimental import pallas as pl
from jax.experimental.pallas import tpu as pltpu
from jax.experimental.pallas import tpu_sc as plsc

NUM_EMB = 1_000_000
DIM = 32
NB = 4096
NS_TOK = 200
TOTAL = NB * NS_TOK

_info = plsc.get_sparse_core_info()
NC = _info.num_cores
NSUB = _info.num_subcores
NW = NC * NSUB
SPAN = TOTAL // NW
CHUNK = 1600
NCHUNK = SPAN // CHUNK


def _emb_body(idx_hbm, tbl_hbm, out_hbm,
              ix0, ix1, g0, g1,
              si0, si1, sg0, sg1, so0, so1):
    ix = (ix0, ix1)
    g = (g0, g1)
    si = (si0, si1)
    sg = (sg0, sg1)
    so = (so0, so1)

    wid = lax.axis_index("s") * NC + lax.axis_index("c")
    base = wid * SPAN

    def start_idx(c, buf):
        pltpu.async_copy(idx_hbm.at[pl.ds(base + c * CHUNK, CHUNK)],
                         ix[buf], si[buf])

    def wait_idx(c, buf):
        pltpu.make_async_copy(idx_hbm.at[pl.ds(base + c * CHUNK, CHUNK)],
                              ix[buf], si[buf]).wait()

    def start_gather(buf):
        pltpu.async_copy(tbl_hbm.at[ix[buf]], g[buf], sg[buf])

    def wait_gather(buf):
        pltpu.make_async_copy(tbl_hbm.at[ix[buf]], g[buf], sg[buf]).wait()

    def start_write(c, buf):
        pltpu.async_copy(g[buf],
                         out_hbm.at[pl.ds(base + c * CHUNK, CHUNK)], so[buf])

    def wait_write(c, buf):
        pltpu.make_async_copy(g[buf],
                              out_hbm.at[pl.ds(base + c * CHUNK, CHUNK)],
                              so[buf]).wait()

    start_idx(0, 0)
    wait_idx(0, 0)
    start_gather(0)
    start_idx(1, 1)
    for c in range(NCHUNK):
        cur = c % 2
        nxt = 1 - cur
        wait_gather(cur)
        if c >= 1:
            wait_write(c - 1, nxt)
        if c + 1 < NCHUNK:
            wait_idx(c + 1, nxt)
            start_gather(nxt)
        start_write(c, cur)
        if c + 2 < NCHUNK:
            start_idx(c + 2, cur)
    wait_write(NCHUNK - 1, (NCHUNK - 1) % 2)


_emb_call = functools.partial(
    pl.kernel,
    mesh=plsc.VectorSubcoreMesh(core_axis_name="c", subcore_axis_name="s"),
    out_type=jax.ShapeDtypeStruct((TOTAL, DIM), jnp.float32),
    scratch_types=[
        pltpu.VMEM((CHUNK,), jnp.int32),
        pltpu.VMEM((CHUNK,), jnp.int32),
        pltpu.VMEM((CHUNK, DIM), jnp.float32),
        pltpu.VMEM((CHUNK, DIM), jnp.float32),
        pltpu.SemaphoreType.DMA,
        pltpu.SemaphoreType.DMA,
        pltpu.SemaphoreType.DMA,
        pltpu.SemaphoreType.DMA,
        pltpu.SemaphoreType.DMA,
        pltpu.SemaphoreType.DMA,
    ],
    compiler_params=pltpu.CompilerParams(use_tc_tiling_on_sc=False),
)(_emb_body)


@jax.jit
def kernel(token_ids, weight):
    idx = token_ids.reshape(-1).astype(jnp.int32)
    out = _emb_call(idx, weight)
    return out.reshape(NB, NS_TOK, DIM)
